# baseline (speedup 1.0000x reference)
import functools
import itertools

import numpy as np
import jax
import jax.numpy as jnp
from jax.experimental import pallas as pl
from jax.experimental.pallas import tpu as pltpu

BT = 32


def _lenet_kernel(xp_ref, w1_ref, w2_ref, w3_ref, b1_ref, b2_ref, b3_ref,
                  m1_ref, m2_ref, m3_ref, sel_ref, wf_ref, bf_ref, o_ref,
                  xw_ref, c1a, c1b, x2_ref, c2_ref, c3_ref, r_ref):
    f32, bf16 = jnp.float32, jnp.bfloat16
    L = xp_ref.shape[-1]

    i = pl.program_id(0)
    par = jax.lax.rem(i, 2)

    def rd(x, d):
        return pltpu.roll(x, (-d) % L, axis=1) if d % L else x

    for q in range(16):
        xw_ref[q * 16:q * 16 + 8, :] = xp_ref[q * 8:(q + 1) * 8, :]
        xw_ref[q * 16 + 8:(q + 1) * 16, :] = jnp.zeros((8, L), bf16)

    for p in range(4):
        pr, ps = p // 2, p % 2
        col = c1a if p % 2 == 0 else c1b
        for u, v in itertools.product(range(2), range(2)):
            r, s = 2 * pr + u, 2 * ps + v
            ci, cb = (r * 4 + s) * 25, (u * 2 + v) * L
            for t in range(25):
                di, dj = t // 5 - 2, t % 5 - 2
                q = ((r + di) % 4) * 4 + (s + dj) % 4
                src = xw_ref[q * 16:(q + 1) * 16, :]
                if 0 <= r + di <= 3 and 0 <= s + dj <= 3:
                    col[t * 16:(t + 1) * 16, cb:cb + L] = src
                else:
                    d = ((r + di) // 4) * 8 + (s + dj) // 4
                    col[t * 16:(t + 1) * 16, cb:cb + L] = rd(src, d) * m1_ref[ci + t]
        y = (jnp.dot(w1_ref[...], col[...], preferred_element_type=f32)
             + b1_ref[...]).astype(bf16)
        x2_ref[pl.ds(par * 128 + p * 32, 32), :] = jnp.maximum(
            jnp.maximum(y[:, 0:L], y[:, L:2 * L]),
            jnp.maximum(y[:, 2 * L:3 * L], y[:, 3 * L:4 * L]))

    for ci in range(4):
        a, b = ci // 2, ci % 2
        cb = ci * L
        for t in range(25):
            di, dj = t // 5 - 2, t % 5 - 2
            q = ((a + di) % 2) * 2 + (b + dj) % 2
            src = x2_ref[pl.ds((1 - par) * 128 + q * 32, 32), :]
            if 0 <= a + di <= 1 and 0 <= b + dj <= 1:
                c2_ref[t * 32:(t + 1) * 32, cb:cb + L] = src
            else:
                d = ((a + di) // 2) * 8 + (b + dj) // 2
                c2_ref[t * 32:(t + 1) * 32, cb:cb + L] = rd(src, d) * m2_ref[ci * 25 + t]
    y2 = (jnp.dot(w2_ref[...], c2_ref[...], preferred_element_type=f32)
          + b2_ref[...]).astype(bf16)
    p2 = jnp.maximum(jnp.maximum(y2[:, 0:L], y2[:, L:2 * L]),
                     jnp.maximum(y2[:, 2 * L:3 * L], y2[:, 3 * L:4 * L]))

    for t in range(25):
        di, dj = t // 5 - 2, t % 5 - 2
        if t == 12:
            c3_ref[t * 32:(t + 1) * 32, :] = p2
        else:
            c3_ref[t * 32:(t + 1) * 32, :] = rd(p2, di * 8 + dj) * m3_ref[t]
    y3 = (jnp.dot(w3_ref[...], c3_ref[...], preferred_element_type=f32)
          + b3_ref[...]).astype(bf16)

    t1 = jnp.maximum(y3, rd(y3, 1))
    t2 = jnp.maximum(t1, rd(t1, 8))

    for k in range(16):
        lk = (k // 4) * 16 + (k % 4) * 2
        r_ref[k * 64:(k + 1) * 64, :] = rd(t2, lk)
    ow = jnp.dot(wf_ref[...], r_ref[...], preferred_element_type=f32)
    o_ref[...] = jnp.dot(ow, sel_ref[...], preferred_element_type=f32) + bf_ref[...]


@functools.lru_cache(maxsize=None)
def _consts(bt):
    L = bt * 64
    lane = np.arange(L) % 64
    h, w = lane // 8, lane % 8

    m1 = np.zeros((400, 1, L), np.float32)
    for ci, (r, s) in enumerate(itertools.product(range(4), range(4))):
        for t in range(25):
            di, dj = t // 5 - 2, t % 5 - 2
            m1[ci * 25 + t, 0] = ((4 * h + r + di >= 0) & (4 * h + r + di < 32)
                                  & (4 * w + s + dj >= 0) & (4 * w + s + dj < 32))

    m2 = np.zeros((100, 1, L), np.float32)
    for ci, (a, b) in enumerate(itertools.product(range(2), range(2))):
        for t in range(25):
            di, dj = t // 5 - 2, t % 5 - 2
            m2[ci * 25 + t, 0] = ((2 * h + a + di >= 0) & (2 * h + a + di < 16)
                                  & (2 * w + b + dj >= 0) & (2 * w + b + dj < 16))

    m3 = np.zeros((25, 1, L), np.float32)
    for t in range(25):
        di, dj = t // 5 - 2, t % 5 - 2
        m3[t, 0] = (h + di >= 0) & (h + di < 8) & (w + dj >= 0) & (w + dj < 8)

    sel = np.zeros((L, bt), np.float32)
    sel[np.arange(bt) * 64, np.arange(bt)] = 1.0
    return (jnp.asarray(m1, jnp.bfloat16), jnp.asarray(m2, jnp.bfloat16),
            jnp.asarray(m3, jnp.bfloat16), jnp.asarray(sel, jnp.float32))


@jax.jit
def _forward(x_nchw, w1, w2, w3, b1, b2, b3, wf, bf):
    B = x_nchw.shape[0]
    g = B // BT
    L = BT * 64

    x = jnp.pad(x_nchw.astype(jnp.bfloat16), ((0, 0), (0, 5), (0, 0), (0, 0)))
    xp = (x.reshape(g, BT, 8, 8, 4, 8, 4)
          .transpose(0, 4, 6, 2, 1, 3, 5)
          .reshape(g, 128, L))

    w1e = jnp.pad(w1.reshape(32, 25, 8), ((0, 0), (0, 0), (0, 8))).reshape(32, 400)

    m1, m2, m3, sel = _consts(BT)

    def full(a):
        return pl.BlockSpec(a.shape, lambda i, _n=a.ndim: (0,) * _n)

    out = pl.pallas_call(
        _lenet_kernel,
        out_shape=jax.ShapeDtypeStruct((g, 10, BT), jnp.float32),
        grid=(g + 1,),
        in_specs=[
            pl.BlockSpec((None, 128, L), lambda i: (jnp.minimum(i, g - 1), 0, 0)),
            full(w1e), full(w2), full(w3), full(b1), full(b2), full(b3),
            full(m1), full(m2), full(m3), full(sel), full(wf), full(bf),
        ],
        out_specs=pl.BlockSpec((None, 10, BT), lambda i: (jnp.maximum(i - 1, 0), 0, 0)),
        scratch_shapes=[
            pltpu.VMEM((256, L), jnp.bfloat16),
            pltpu.VMEM((400, 4 * L), jnp.bfloat16),
            pltpu.VMEM((400, 4 * L), jnp.bfloat16),
            pltpu.VMEM((256, L), jnp.bfloat16),
            pltpu.VMEM((800, 4 * L), jnp.bfloat16),
            pltpu.VMEM((800, L), jnp.bfloat16),
            pltpu.VMEM((1024, L), jnp.bfloat16),
        ],
        compiler_params=pltpu.CompilerParams(
            dimension_semantics=("arbitrary",)),
    )(xp, w1e, w2, w3, b1, b2, b3, m1, m2, m3, sel, wf, bf)

    return out.transpose(0, 2, 1).reshape(B, 10)


def kernel(x_nchw, w1, w2, w3, b1, b2, b3, m1, m2, m3, s1, s2, sg, wf, bf):
    del m1, m2, m3, s1, s2, sg
    return _forward(x_nchw, w1, w2, w3, b1, b2, b3, wf, bf)


# speedup vs baseline: 1.0853x; 1.0853x over previous
import functools
import itertools

import numpy as np
import jax
import jax.numpy as jnp
from jax.experimental import pallas as pl
from jax.experimental.pallas import tpu as pltpu

BT = 64


def _lenet_kernel(xp_ref, w1_ref, w2_ref, w3_ref, b1_ref, b2_ref, b3_ref,
                  m1_ref, m2_ref, m3_ref, sel_ref, wf_ref, bf_ref, o_ref,
                  xw_ref, c1a, c1b, x2_ref, c2_ref, r_ref):
    f32, bf16 = jnp.float32, jnp.bfloat16
    L = xp_ref.shape[-1]

    def rd(x, d):
        return pltpu.roll(x, (-d) % L, axis=1) if d % L else x

    for q in range(16):
        xw_ref[q * 16:q * 16 + 8, :] = xp_ref[q * 8:(q + 1) * 8, :]
        xw_ref[q * 16 + 8:(q + 1) * 16, :] = jnp.zeros((8, L), bf16)

    for p in range(4):
        pr, ps = p // 2, p % 2
        col = c1a if p % 2 == 0 else c1b
        for u, v in itertools.product(range(2), range(2)):
            r, s = 2 * pr + u, 2 * ps + v
            ci, cb = (r * 4 + s) * 25, (u * 2 + v) * L
            for t in range(25):
                di, dj = t // 5 - 2, t % 5 - 2
                q = ((r + di) % 4) * 4 + (s + dj) % 4
                src = xw_ref[q * 16:(q + 1) * 16, :]
                if 0 <= r + di <= 3 and 0 <= s + dj <= 3:
                    col[t * 16:(t + 1) * 16, cb:cb + L] = src
                else:
                    d = ((r + di) // 4) * 8 + (s + dj) // 4
                    col[t * 16:(t + 1) * 16, cb:cb + L] = rd(src, d) * m1_ref[ci + t]
        y = (jnp.dot(w1_ref[...], col[...], preferred_element_type=f32)
             + b1_ref[...]).astype(bf16)
        x2_ref[p * 32:(p + 1) * 32, :] = jnp.maximum(
            jnp.maximum(y[:, 0:L], y[:, L:2 * L]),
            jnp.maximum(y[:, 2 * L:3 * L], y[:, 3 * L:4 * L]))

    p2 = None
    for half in range(2):
        for k in range(2):
            ci = 2 * half + k
            a, b = ci // 2, ci % 2
            cb = k * L
            for t in range(25):
                di, dj = t // 5 - 2, t % 5 - 2
                q = ((a + di) % 2) * 2 + (b + dj) % 2
                src = x2_ref[q * 32:(q + 1) * 32, :]
                if 0 <= a + di <= 1 and 0 <= b + dj <= 1:
                    c2_ref[t * 32:(t + 1) * 32, cb:cb + L] = src
                else:
                    d = ((a + di) // 2) * 8 + (b + dj) // 2
                    c2_ref[t * 32:(t + 1) * 32, cb:cb + L] = rd(src, d) * m2_ref[ci * 25 + t]
        y2 = (jnp.dot(w2_ref[...], c2_ref[...], preferred_element_type=f32)
              + b2_ref[...]).astype(bf16)
        h = jnp.maximum(y2[:, 0:L], y2[:, L:2 * L])
        p2 = h if half == 0 else jnp.maximum(p2, h)

    c3_ref = c2_ref
    for t in range(25):
        di, dj = t // 5 - 2, t % 5 - 2
        if t == 12:
            c3_ref[t * 32:(t + 1) * 32, 0:L] = p2
        else:
            c3_ref[t * 32:(t + 1) * 32, 0:L] = rd(p2, di * 8 + dj) * m3_ref[t]
    y3 = (jnp.dot(w3_ref[...], c3_ref[0:800, 0:L], preferred_element_type=f32)
          + b3_ref[...]).astype(bf16)

    t1 = jnp.maximum(y3, rd(y3, 1))
    t2 = jnp.maximum(t1, rd(t1, 8))

    for k in range(16):
        lk = (k // 4) * 16 + (k % 4) * 2
        r_ref[k * 64:(k + 1) * 64, :] = rd(t2, lk)
    ow = jnp.dot(wf_ref[...], r_ref[...], preferred_element_type=f32)
    o_ref[...] = jnp.dot(ow, sel_ref[...], preferred_element_type=f32) + bf_ref[...]


@functools.lru_cache(maxsize=None)
def _consts(bt):
    L = bt * 64
    lane = np.arange(L) % 64
    h, w = lane // 8, lane % 8

    m1 = np.zeros((400, 1, L), np.float32)
    for ci, (r, s) in enumerate(itertools.product(range(4), range(4))):
        for t in range(25):
            di, dj = t // 5 - 2, t % 5 - 2
            m1[ci * 25 + t, 0] = ((4 * h + r + di >= 0) & (4 * h + r + di < 32)
                                  & (4 * w + s + dj >= 0) & (4 * w + s + dj < 32))

    m2 = np.zeros((100, 1, L), np.float32)
    for ci, (a, b) in enumerate(itertools.product(range(2), range(2))):
        for t in range(25):
            di, dj = t // 5 - 2, t % 5 - 2
            m2[ci * 25 + t, 0] = ((2 * h + a + di >= 0) & (2 * h + a + di < 16)
                                  & (2 * w + b + dj >= 0) & (2 * w + b + dj < 16))

    m3 = np.zeros((25, 1, L), np.float32)
    for t in range(25):
        di, dj = t // 5 - 2, t % 5 - 2
        m3[t, 0] = (h + di >= 0) & (h + di < 8) & (w + dj >= 0) & (w + dj < 8)

    sel = np.zeros((L, bt), np.float32)
    sel[np.arange(bt) * 64, np.arange(bt)] = 1.0
    return (jnp.asarray(m1, jnp.bfloat16), jnp.asarray(m2, jnp.bfloat16),
            jnp.asarray(m3, jnp.bfloat16), jnp.asarray(sel, jnp.float32))


@jax.jit
def _forward(x_nchw, w1, w2, w3, b1, b2, b3, wf, bf):
    B = x_nchw.shape[0]
    g = B // BT
    L = BT * 64

    x = jnp.pad(x_nchw.astype(jnp.bfloat16), ((0, 0), (0, 5), (0, 0), (0, 0)))
    xp = (x.reshape(g, BT, 8, 8, 4, 8, 4)
          .transpose(0, 4, 6, 2, 1, 3, 5)
          .reshape(g, 128, L))

    w1e = jnp.pad(w1.reshape(32, 25, 8), ((0, 0), (0, 0), (0, 8))).reshape(32, 400)

    m1, m2, m3, sel = _consts(BT)

    def full(a):
        return pl.BlockSpec(a.shape, lambda i, _n=a.ndim: (0,) * _n)

    out = pl.pallas_call(
        _lenet_kernel,
        out_shape=jax.ShapeDtypeStruct((g, 10, BT), jnp.float32),
        grid=(g,),
        in_specs=[
            pl.BlockSpec((None, 128, L), lambda i: (i, 0, 0)),
            full(w1e), full(w2), full(w3), full(b1), full(b2), full(b3),
            full(m1), full(m2), full(m3), full(sel), full(wf), full(bf),
        ],
        out_specs=pl.BlockSpec((None, 10, BT), lambda i: (i, 0, 0)),
        scratch_shapes=[
            pltpu.VMEM((256, L), jnp.bfloat16),
            pltpu.VMEM((400, 4 * L), jnp.bfloat16),
            pltpu.VMEM((400, 4 * L), jnp.bfloat16),
            pltpu.VMEM((128, L), jnp.bfloat16),
            pltpu.VMEM((800, 2 * L), jnp.bfloat16),
            pltpu.VMEM((1024, L), jnp.bfloat16),
        ],
        compiler_params=pltpu.CompilerParams(
            dimension_semantics=("parallel",)),
    )(xp, w1e, w2, w3, b1, b2, b3, m1, m2, m3, sel, wf, bf)

    return out.transpose(0, 2, 1).reshape(B, 10)


def kernel(x_nchw, w1, w2, w3, b1, b2, b3, m1, m2, m3, s1, s2, sg, wf, bf):
    del m1, m2, m3, s1, s2, sg
    return _forward(x_nchw, w1, w2, w3, b1, b2, b3, wf, bf)


# speedup vs baseline: 1.1895x; 1.0960x over previous
import functools
import itertools

import numpy as np
import jax
import jax.numpy as jnp
from jax.experimental import pallas as pl
from jax.experimental.pallas import tpu as pltpu

BT = 64


def _lenet_kernel(xp_ref, w1_ref, w2_ref, w3_ref, b1_ref, b2_ref, b3_ref,
                  m1_ref, m2_ref, m3_ref, sel_ref, wf_ref, bf_ref, o_ref,
                  xw_ref, c1a, c1b, x2_ref, c2_ref, r_ref):
    f32, bf16 = jnp.float32, jnp.bfloat16
    L = xp_ref.shape[-1]

    def rd(x, d):
        return pltpu.roll(x, (-d) % L, axis=1) if d % L else x

    for q in range(16):
        xw_ref[q * 16:q * 16 + 4, :] = xp_ref[q * 4:(q + 1) * 4, :]
        xw_ref[q * 16 + 4:(q + 1) * 16, :] = jnp.zeros((12, L), bf16)

    for p in range(4):
        pr, ps = p // 2, p % 2
        col = c1a if p % 2 == 0 else c1b
        for u, v in itertools.product(range(2), range(2)):
            r, s = 2 * pr + u, 2 * ps + v
            ci, cb = (r * 4 + s) * 25, (u * 2 + v) * L
            for t in range(25):
                di, dj = t // 5 - 2, t % 5 - 2
                q = ((r + di) % 4) * 4 + (s + dj) % 4
                src = xw_ref[q * 16:(q + 1) * 16, :]
                if 0 <= r + di <= 3 and 0 <= s + dj <= 3:
                    col[t * 16:(t + 1) * 16, cb:cb + L] = src
                else:
                    d = ((r + di) // 4) * 8 + (s + dj) // 4
                    col[t * 16:(t + 1) * 16, cb:cb + L] = rd(src, d) * m1_ref[ci + t]
        y = (jnp.dot(w1_ref[...], col[...], preferred_element_type=f32)
             + b1_ref[...]).astype(bf16)
        x2_ref[p * 32:(p + 1) * 32, :] = jnp.maximum(
            jnp.maximum(y[:, 0:L], y[:, L:2 * L]),
            jnp.maximum(y[:, 2 * L:3 * L], y[:, 3 * L:4 * L]))

    p2 = None
    for half in range(2):
        for k in range(2):
            ci = 2 * half + k
            a, b = ci // 2, ci % 2
            cb = k * L
            for t in range(25):
                di, dj = t // 5 - 2, t % 5 - 2
                q = ((a + di) % 2) * 2 + (b + dj) % 2
                src = x2_ref[q * 32:(q + 1) * 32, :]
                if 0 <= a + di <= 1 and 0 <= b + dj <= 1:
                    c2_ref[t * 32:(t + 1) * 32, cb:cb + L] = src
                else:
                    d = ((a + di) // 2) * 8 + (b + dj) // 2
                    c2_ref[t * 32:(t + 1) * 32, cb:cb + L] = rd(src, d) * m2_ref[ci * 25 + t]
        y2 = (jnp.dot(w2_ref[...], c2_ref[...], preferred_element_type=f32)
              + b2_ref[...]).astype(bf16)
        h = jnp.maximum(y2[:, 0:L], y2[:, L:2 * L])
        p2 = h if half == 0 else jnp.maximum(p2, h)

    c3_ref = c2_ref
    for t in range(25):
        di, dj = t // 5 - 2, t % 5 - 2
        if t == 12:
            c3_ref[t * 32:(t + 1) * 32, 0:L] = p2
        else:
            c3_ref[t * 32:(t + 1) * 32, 0:L] = rd(p2, di * 8 + dj) * m3_ref[t]
    y3 = (jnp.dot(w3_ref[...], c3_ref[0:800, 0:L], preferred_element_type=f32)
          + b3_ref[...]).astype(bf16)

    t1 = jnp.maximum(y3, rd(y3, 1))
    t2 = jnp.maximum(t1, rd(t1, 8))

    for k in range(16):
        lk = (k // 4) * 16 + (k % 4) * 2
        r_ref[k * 64:(k + 1) * 64, :] = rd(t2, lk)
    ow = jnp.dot(wf_ref[...], r_ref[...], preferred_element_type=f32)
    o_ref[...] = jnp.dot(ow, sel_ref[...], preferred_element_type=f32) + bf_ref[...]


@functools.lru_cache(maxsize=None)
def _consts(bt):
    L = bt * 64
    lane = np.arange(L) % 64
    h, w = lane // 8, lane % 8

    m1 = np.zeros((400, 1, L), np.float32)
    for ci, (r, s) in enumerate(itertools.product(range(4), range(4))):
        for t in range(25):
            di, dj = t // 5 - 2, t % 5 - 2
            m1[ci * 25 + t, 0] = ((4 * h + r + di >= 0) & (4 * h + r + di < 32)
                                  & (4 * w + s + dj >= 0) & (4 * w + s + dj < 32))

    m2 = np.zeros((100, 1, L), np.float32)
    for ci, (a, b) in enumerate(itertools.product(range(2), range(2))):
        for t in range(25):
            di, dj = t // 5 - 2, t % 5 - 2
            m2[ci * 25 + t, 0] = ((2 * h + a + di >= 0) & (2 * h + a + di < 16)
                                  & (2 * w + b + dj >= 0) & (2 * w + b + dj < 16))

    m3 = np.zeros((25, 1, L), np.float32)
    for t in range(25):
        di, dj = t // 5 - 2, t % 5 - 2
        m3[t, 0] = (h + di >= 0) & (h + di < 8) & (w + dj >= 0) & (w + dj < 8)

    sel = np.zeros((L, bt), np.float32)
    sel[np.arange(bt) * 64, np.arange(bt)] = 1.0
    return (jnp.asarray(m1, jnp.bfloat16), jnp.asarray(m2, jnp.bfloat16),
            jnp.asarray(m3, jnp.bfloat16), jnp.asarray(sel, jnp.float32))


@jax.jit
def _forward(x_nchw, w1, w2, w3, b1, b2, b3, wf, bf):
    B = x_nchw.shape[0]
    g = B // BT
    L = BT * 64

    x = jnp.pad(x_nchw.astype(jnp.bfloat16), ((0, 0), (0, 1), (0, 0), (0, 0)))
    xp = (x.reshape(g, BT, 4, 8, 4, 8, 4)
          .transpose(0, 4, 6, 2, 1, 3, 5)
          .reshape(g, 64, L))

    w1e = jnp.pad(w1.reshape(32, 25, 8), ((0, 0), (0, 0), (0, 8))).reshape(32, 400)

    m1, m2, m3, sel = _consts(BT)

    def full(a):
        return pl.BlockSpec(a.shape, lambda i, _n=a.ndim: (0,) * _n)

    out = pl.pallas_call(
        _lenet_kernel,
        out_shape=jax.ShapeDtypeStruct((g, 10, BT), jnp.float32),
        grid=(g,),
        in_specs=[
            pl.BlockSpec((None, 64, L), lambda i: (i, 0, 0)),
            full(w1e), full(w2), full(w3), full(b1), full(b2), full(b3),
            full(m1), full(m2), full(m3), full(sel), full(wf), full(bf),
        ],
        out_specs=pl.BlockSpec((None, 10, BT), lambda i: (i, 0, 0)),
        scratch_shapes=[
            pltpu.VMEM((256, L), jnp.bfloat16),
            pltpu.VMEM((400, 4 * L), jnp.bfloat16),
            pltpu.VMEM((400, 4 * L), jnp.bfloat16),
            pltpu.VMEM((128, L), jnp.bfloat16),
            pltpu.VMEM((800, 2 * L), jnp.bfloat16),
            pltpu.VMEM((1024, L), jnp.bfloat16),
        ],
        compiler_params=pltpu.CompilerParams(
            dimension_semantics=("parallel",)),
    )(xp, w1e, w2, w3, b1, b2, b3, m1, m2, m3, sel, wf, bf)

    return out.transpose(0, 2, 1).reshape(B, 10)


def kernel(x_nchw, w1, w2, w3, b1, b2, b3, m1, m2, m3, s1, s2, sg, wf, bf):
    del m1, m2, m3, s1, s2, sg
    return _forward(x_nchw, w1, w2, w3, b1, b2, b3, wf, bf)


# speedup vs baseline: 1.1907x; 1.0010x over previous
import functools
import itertools

import numpy as np
import jax
import jax.numpy as jnp
from jax.experimental import pallas as pl
from jax.experimental.pallas import tpu as pltpu

BT = 64


def _lenet_kernel(xp_ref, w1_ref, w2_ref, w3_ref, b1_ref, b2_ref, b3_ref,
                  m1_ref, m2_ref, m3_ref, sel_ref, wf_ref, bf_ref, o_ref,
                  xw_ref, c1a, c1b, x2_ref, c2_ref, r_ref):
    f32, bf16 = jnp.float32, jnp.bfloat16
    L = xp_ref.shape[-1]

    def rd(x, d):
        return pltpu.roll(x, (-d) % L, axis=1) if d % L else x

    for q in range(16):
        xw_ref[q * 16:q * 16 + 4, :] = xp_ref[q * 4:(q + 1) * 4, :]
        xw_ref[q * 16 + 4:(q + 1) * 16, :] = jnp.zeros((12, L), bf16)

    for p in range(4):
        pr, ps = p // 2, p % 2
        col = c1a if p % 2 == 0 else c1b
        for u, v in itertools.product(range(2), range(2)):
            r, s = 2 * pr + u, 2 * ps + v
            ci, cb = (r * 4 + s) * 25, (u * 2 + v) * L
            for t in range(25):
                di, dj = t // 5 - 2, t % 5 - 2
                q = ((r + di) % 4) * 4 + (s + dj) % 4
                src = xw_ref[q * 16:(q + 1) * 16, :]
                if 0 <= r + di <= 3 and 0 <= s + dj <= 3:
                    col[t * 16:(t + 1) * 16, cb:cb + L] = src
                else:
                    d = ((r + di) // 4) * 8 + (s + dj) // 4
                    col[t * 16:(t + 1) * 16, cb:cb + L] = rd(src, d) * m1_ref[ci + t]
        y = (jnp.dot(w1_ref[...], col[...], preferred_element_type=f32)
             + b1_ref[...]).astype(bf16)
        x2_ref[p * 32:(p + 1) * 32, :] = jnp.maximum(
            jnp.maximum(y[:, 0:L], y[:, L:2 * L]),
            jnp.maximum(y[:, 2 * L:3 * L], y[:, 3 * L:4 * L]))

    p2 = None
    for half in range(2):
        for k in range(2):
            ci = 2 * half + k
            a, b = ci // 2, ci % 2
            cb = k * L
            for t in range(25):
                di, dj = t // 5 - 2, t % 5 - 2
                q = ((a + di) % 2) * 2 + (b + dj) % 2
                src = x2_ref[q * 32:(q + 1) * 32, :]
                if 0 <= a + di <= 1 and 0 <= b + dj <= 1:
                    c2_ref[t * 32:(t + 1) * 32, cb:cb + L] = src
                else:
                    d = ((a + di) // 2) * 8 + (b + dj) // 2
                    c2_ref[t * 32:(t + 1) * 32, cb:cb + L] = rd(src, d) * m2_ref[ci * 25 + t]
        y2 = (jnp.dot(w2_ref[...], c2_ref[...], preferred_element_type=f32)
              + b2_ref[...]).astype(bf16)
        h = jnp.maximum(y2[:, 0:L], y2[:, L:2 * L])
        p2 = h if half == 0 else jnp.maximum(p2, h)

    c3_ref = c2_ref
    for t in range(25):
        di, dj = t // 5 - 2, t % 5 - 2
        if t == 12:
            c3_ref[t * 32:(t + 1) * 32, 0:L] = p2
        else:
            c3_ref[t * 32:(t + 1) * 32, 0:L] = rd(p2, di * 8 + dj) * m3_ref[t]
    y3 = (jnp.dot(w3_ref[...], c3_ref[0:800, 0:L], preferred_element_type=f32)
          + b3_ref[...]).astype(bf16)

    t1 = jnp.maximum(y3, rd(y3, 1))
    t2 = jnp.maximum(t1, rd(t1, 8))

    for k in range(16):
        lk = (k // 4) * 16 + (k % 4) * 2
        r_ref[k * 64:(k + 1) * 64, :] = rd(t2, lk)
    ow = jnp.dot(wf_ref[...], r_ref[...], preferred_element_type=f32)
    o_ref[...] = jnp.dot(ow, sel_ref[...], preferred_element_type=f32) + bf_ref[...]


@functools.lru_cache(maxsize=None)
def _consts(bt):
    L = bt * 64
    lane = np.arange(L) % 64
    h, w = lane // 8, lane % 8

    m1 = np.zeros((400, 1, L), np.float32)
    for ci, (r, s) in enumerate(itertools.product(range(4), range(4))):
        for t in range(25):
            di, dj = t // 5 - 2, t % 5 - 2
            m1[ci * 25 + t, 0] = ((4 * h + r + di >= 0) & (4 * h + r + di < 32)
                                  & (4 * w + s + dj >= 0) & (4 * w + s + dj < 32))

    m2 = np.zeros((100, 1, L), np.float32)
    for ci, (a, b) in enumerate(itertools.product(range(2), range(2))):
        for t in range(25):
            di, dj = t // 5 - 2, t % 5 - 2
            m2[ci * 25 + t, 0] = ((2 * h + a + di >= 0) & (2 * h + a + di < 16)
                                  & (2 * w + b + dj >= 0) & (2 * w + b + dj < 16))

    m3 = np.zeros((25, 1, L), np.float32)
    for t in range(25):
        di, dj = t // 5 - 2, t % 5 - 2
        m3[t, 0] = (h + di >= 0) & (h + di < 8) & (w + dj >= 0) & (w + dj < 8)

    sel = np.zeros((L, bt), np.float32)
    sel[np.arange(bt) * 64, np.arange(bt)] = 1.0
    return (jnp.asarray(m1, jnp.bfloat16), jnp.asarray(m2, jnp.bfloat16),
            jnp.asarray(m3, jnp.bfloat16), jnp.asarray(sel, jnp.float32))


@jax.jit
def _forward(x_nchw, w1, w2, w3, b1, b2, b3, wf, bf):
    B = x_nchw.shape[0]
    g = B // BT
    L = BT * 64

    x = jnp.pad(x_nchw.astype(jnp.bfloat16), ((0, 0), (0, 1), (0, 0), (0, 0)))
    xp = (x.reshape(g, BT, 4, 8, 4, 8, 4)
          .transpose(0, 4, 6, 2, 1, 3, 5)
          .reshape(g, 64, L))

    w1e = jnp.pad(w1.reshape(32, 25, 8), ((0, 0), (0, 0), (0, 8))).reshape(32, 400)

    m1, m2, m3, sel = _consts(BT)

    def full(a):
        return pl.BlockSpec(a.shape, lambda i, _n=a.ndim: (0,) * _n)

    out = pl.pallas_call(
        _lenet_kernel,
        out_shape=jax.ShapeDtypeStruct((g, 10, BT), jnp.float32),
        grid=(g,),
        in_specs=[
            pl.BlockSpec((None, 64, L), lambda i: (i, 0, 0)),
            full(w1e), full(w2), full(w3), full(b1), full(b2), full(b3),
            full(m1), full(m2), full(m3), full(sel), full(wf), full(bf),
        ],
        out_specs=pl.BlockSpec((None, 10, BT), lambda i: (i, 0, 0)),
        scratch_shapes=[
            pltpu.VMEM((256, L), jnp.bfloat16),
            pltpu.VMEM((400, 4 * L), jnp.bfloat16),
            pltpu.VMEM((400, 4 * L), jnp.bfloat16),
            pltpu.VMEM((128, L), jnp.bfloat16),
            pltpu.VMEM((800, 2 * L), jnp.bfloat16),
            pltpu.VMEM((1024, L), jnp.bfloat16),
        ],
        compiler_params=pltpu.CompilerParams(
            dimension_semantics=("arbitrary",)),
    )(xp, w1e, w2, w3, b1, b2, b3, m1, m2, m3, sel, wf, bf)

    return out.transpose(0, 2, 1).reshape(B, 10)


def kernel(x_nchw, w1, w2, w3, b1, b2, b3, m1, m2, m3, s1, s2, sg, wf, bf):
    del m1, m2, m3, s1, s2, sg
    return _forward(x_nchw, w1, w2, w3, b1, b2, b3, wf, bf)
